# baseline (speedup 1.0000x reference)
import functools

import jax
import jax.numpy as jnp
from jax import lax
from jax.experimental import pallas as pl
from jax.experimental.pallas import tpu as pltpu
from jax.experimental.pallas import tpu_sc as plsc

NC = 2
NS = 16
NW = NC * NS
SUB = 128
RC = 16
MEGA = 160
CHUNK = SUB * MEGA
NPAD = 102400
RPT = NPAD // NS
TBL = 2048

_SC_PARAMS = pltpu.CompilerParams(use_tc_tiling_on_sc=False)
_MESH = dict(
    mesh=plsc.VectorSubcoreMesh(core_axis_name="c", subcore_axis_name="s"),
    compiler_params=_SC_PARAMS,
)


def _deg_body(m_chunks, comb, zeros1, ones, deg_out,
              idx_v, ones_v, stage_v, deg_sh, ssem):
    c = lax.axis_index("c")
    s = lax.axis_index("s")
    sl = pl.ds(s * RPT, RPT)
    pltpu.sync_copy(zeros1.at[sl], stage_v)
    pltpu.sync_copy(stage_v, deg_sh.at[sl])
    pltpu.sync_copy(ones, ones_v)
    plsc.subcore_barrier()
    w = c * NS + s

    def mega(m, carry):
        pltpu.sync_copy(comb.at[w, m, 1], idx_v)

        def burst(b, cc):
            for r in range(RC):
                pltpu.async_copy(ones_v, deg_sh.at[idx_v.at[b * RC + r]],
                                 ssem, add=True)
            for r in range(RC):
                pltpu.make_async_copy(ones_v,
                                      deg_sh.at[idx_v.at[b * RC + r]],
                                      ssem).wait()
            return cc

        lax.fori_loop(0, MEGA // RC, burst, 0)
        return carry

    lax.fori_loop(0, m_chunks, mega, 0)
    plsc.subcore_barrier()
    pltpu.sync_copy(deg_sh.at[sl], stage_v)
    pltpu.sync_copy(stage_v, deg_out.at[pl.ds(c * NPAD + s * RPT, RPT)])


def _agg_body(m_chunks, tab0, tab1, comb, zeros1,
              out0, out1, idx_v, rows0, rows1, stage_v,
              tab0_sh, tab1_sh, acc0_sh, acc1_sh, gsem, ssem):
    c = lax.axis_index("c")
    s = lax.axis_index("s")
    sl = pl.ds(s * RPT, RPT)
    pltpu.sync_copy(tab0.at[sl], stage_v)
    pltpu.sync_copy(stage_v, tab0_sh.at[sl])
    pltpu.sync_copy(tab1.at[sl], stage_v)
    pltpu.sync_copy(stage_v, tab1_sh.at[sl])
    pltpu.sync_copy(zeros1.at[sl], stage_v)
    pltpu.sync_copy(stage_v, acc0_sh.at[sl])
    pltpu.sync_copy(stage_v, acc1_sh.at[sl])
    plsc.subcore_barrier()
    w = c * NS + s

    def mega(m, carry):
        pltpu.sync_copy(comb.at[w, m], idx_v)

        def burst(b, cc):
            for r in range(RC):
                pltpu.async_copy(tab0_sh.at[idx_v.at[0, b * RC + r]],
                                 rows0.at[r], gsem)
                pltpu.async_copy(tab1_sh.at[idx_v.at[0, b * RC + r]],
                                 rows1.at[r], gsem)
            for r in range(RC):
                pltpu.make_async_copy(tab0_sh.at[idx_v.at[0, b * RC + r]],
                                      rows0.at[r], gsem).wait()
                pltpu.make_async_copy(tab1_sh.at[idx_v.at[0, b * RC + r]],
                                      rows1.at[r], gsem).wait()
            for r in range(RC):
                pltpu.async_copy(rows0.at[r],
                                 acc0_sh.at[idx_v.at[1, b * RC + r]],
                                 ssem, add=True)
                pltpu.async_copy(rows1.at[r],
                                 acc1_sh.at[idx_v.at[1, b * RC + r]],
                                 ssem, add=True)
            for r in range(RC):
                pltpu.make_async_copy(rows0.at[r],
                                      acc0_sh.at[idx_v.at[1, b * RC + r]],
                                      ssem).wait()
                pltpu.make_async_copy(rows1.at[r],
                                      acc1_sh.at[idx_v.at[1, b * RC + r]],
                                      ssem).wait()
            return cc

        lax.fori_loop(0, MEGA // RC, burst, 0)
        return carry

    lax.fori_loop(0, m_chunks, mega, 0)
    plsc.subcore_barrier()
    pltpu.sync_copy(acc0_sh.at[sl], stage_v)
    pltpu.sync_copy(stage_v, out0.at[pl.ds(c * NPAD + s * RPT, RPT)])
    pltpu.sync_copy(acc1_sh.at[sl], stage_v)
    pltpu.sync_copy(stage_v, out1.at[pl.ds(c * NPAD + s * RPT, RPT)])


def _sc_deg(comb, zeros1, ones, m_chunks):
    return pl.kernel(
        functools.partial(_deg_body, m_chunks),
        out_type=jax.ShapeDtypeStruct((NC * NPAD,), jnp.float32),
        scratch_types=[
            pltpu.VMEM((MEGA, SUB), jnp.int32),
            pltpu.VMEM((SUB,), jnp.float32),
            pltpu.VMEM((RPT,), jnp.float32),
            pltpu.VMEM_SHARED((NPAD,), jnp.float32),
            pltpu.SemaphoreType.DMA,
        ],
        **_MESH)(comb, zeros1, ones)


def _sc_agg(tab0, tab1, comb, zeros1, m_chunks):
    return pl.kernel(
        functools.partial(_agg_body, m_chunks),
        out_type=[
            jax.ShapeDtypeStruct((NC * NPAD,), jnp.float32),
            jax.ShapeDtypeStruct((NC * NPAD,), jnp.float32),
        ],
        scratch_types=[
            pltpu.VMEM((2, MEGA, SUB), jnp.int32),
            pltpu.VMEM((RC, SUB), jnp.float32),
            pltpu.VMEM((RC, SUB), jnp.float32),
            pltpu.VMEM((RPT,), jnp.float32),
            pltpu.VMEM_SHARED((NPAD,), jnp.float32),
            pltpu.VMEM_SHARED((NPAD,), jnp.float32),
            pltpu.VMEM_SHARED((NPAD,), jnp.float32),
            pltpu.VMEM_SHARED((NPAD,), jnp.float32),
            pltpu.SemaphoreType.DMA,
            pltpu.SemaphoreType.DMA,
        ],
        **_MESH)(tab0, tab1, comb, zeros1)


def _t1_body(degp, xt, dis, xs):
    deg = degp[0:1, :] + degp[1:2, :] + 1.0
    d = lax.rsqrt(deg)
    dis[...] = d
    xs[...] = xt[...] * d


def _t2_body(n, a0, a1, xs, dis, w1t, b1c, w2t, ys):
    i = pl.program_id(0)
    agg0 = a0[0:1, :] + a0[1:2, :] + xs[0:1, :]
    agg1 = a1[0:1, :] + a1[1:2, :] + xs[1:2, :]
    pre = jnp.concatenate([agg0, agg1], axis=0) * dis[...]
    h = jnp.dot(w1t[...], pre, preferred_element_type=jnp.float32) + b1c[...]
    h = jnp.maximum(h, 0.0)
    y = jnp.dot(w2t[...], h, preferred_element_type=jnp.float32)
    out = y * dis[...]
    col = i * TBL + lax.broadcasted_iota(jnp.int32, (2, TBL), 1)
    ys[...] = jnp.where(col < n, out, 0.0)


def _t3_body(a0, a1, ys, dis, b2c, out):
    v0 = a0[0:1, :] + a0[1:2, :] + ys[0:1, :]
    v1 = a1[0:1, :] + a1[1:2, :] + ys[1:2, :]
    v = jnp.concatenate([v0, v1], axis=0) * dis[...] + b2c[...]
    m = jnp.max(v, axis=0, keepdims=True)
    lse = m + jnp.log(jnp.sum(jnp.exp(v - m), axis=0, keepdims=True))
    out[...] = v - lse


def kernel(x, edge_index, W1, b1, W2, b2):
    n = x.shape[0]
    e = edge_index.shape[1]
    f_hid = W1.shape[1]
    assert n <= NPAD - 1 and NPAD % (NS * 8) == 0

    src = edge_index[0].astype(jnp.int32)
    dst = edge_index[1].astype(jnp.int32)
    step = NW * CHUNK
    epad = ((e + step - 1) // step) * step
    if epad != e:
        fill = jnp.full((epad - e,), n, jnp.int32)
        src = jnp.concatenate([src, fill])
        dst = jnp.concatenate([dst, fill])
    rows_w = epad // NW // SUB
    m_chunks = rows_w // MEGA
    comb = jnp.stack(
        [src.reshape(NW, m_chunks, MEGA, SUB),
         dst.reshape(NW, m_chunks, MEGA, SUB)], axis=2)

    xt = jnp.zeros((2, NPAD), jnp.float32).at[:, :n].set(x.T)
    zeros1 = jnp.zeros((NPAD,), jnp.float32)
    ones = jnp.ones((SUB,), jnp.float32)

    degp = _sc_deg(comb, zeros1, ones, m_chunks)
    degp = degp.reshape(NC, NPAD)

    grid = NPAD // TBL
    dis, xs = pl.pallas_call(
        _t1_body,
        grid=(grid,),
        in_specs=[
            pl.BlockSpec((NC, TBL), lambda i: (0, i)),
            pl.BlockSpec((2, TBL), lambda i: (0, i)),
        ],
        out_specs=[
            pl.BlockSpec((1, TBL), lambda i: (0, i)),
            pl.BlockSpec((2, TBL), lambda i: (0, i)),
        ],
        out_shape=[
            jax.ShapeDtypeStruct((1, NPAD), jnp.float32),
            jax.ShapeDtypeStruct((2, NPAD), jnp.float32),
        ],
    )(degp, xt)

    a10, a11 = _sc_agg(xs[0], xs[1], comb, zeros1, m_chunks)

    ys = pl.pallas_call(
        functools.partial(_t2_body, n),
        grid=(grid,),
        in_specs=[
            pl.BlockSpec((NC, TBL), lambda i: (0, i)),
            pl.BlockSpec((NC, TBL), lambda i: (0, i)),
            pl.BlockSpec((2, TBL), lambda i: (0, i)),
            pl.BlockSpec((1, TBL), lambda i: (0, i)),
            pl.BlockSpec((f_hid, 2), lambda i: (0, 0)),
            pl.BlockSpec((f_hid, 1), lambda i: (0, 0)),
            pl.BlockSpec((2, f_hid), lambda i: (0, 0)),
        ],
        out_specs=pl.BlockSpec((2, TBL), lambda i: (0, i)),
        out_shape=jax.ShapeDtypeStruct((2, NPAD), jnp.float32),
    )(a10.reshape(NC, NPAD), a11.reshape(NC, NPAD), xs, dis,
      W1.T, b1.reshape(f_hid, 1), W2.T)

    a20, a21 = _sc_agg(ys[0], ys[1], comb, zeros1, m_chunks)

    out_t = pl.pallas_call(
        _t3_body,
        grid=(grid,),
        in_specs=[
            pl.BlockSpec((NC, TBL), lambda i: (0, i)),
            pl.BlockSpec((NC, TBL), lambda i: (0, i)),
            pl.BlockSpec((2, TBL), lambda i: (0, i)),
            pl.BlockSpec((1, TBL), lambda i: (0, i)),
            pl.BlockSpec((2, 1), lambda i: (0, 0)),
        ],
        out_specs=pl.BlockSpec((2, TBL), lambda i: (0, i)),
        out_shape=jax.ShapeDtypeStruct((2, NPAD), jnp.float32),
    )(a20.reshape(NC, NPAD), a21.reshape(NC, NPAD), ys, dis,
      b2.reshape(2, 1))

    return out_t[:, :n].T


# speedup vs baseline: 1.0128x; 1.0128x over previous
import functools

import jax
import jax.numpy as jnp
from jax import lax
from jax.experimental import pallas as pl
from jax.experimental.pallas import tpu as pltpu
from jax.experimental.pallas import tpu_sc as plsc

NC = 2
NS = 16
NW = NC * NS
CB = 12800
NPAD = 102400
RPT = NPAD // NS
TBL = 2048

_SC_PARAMS = pltpu.CompilerParams(use_tc_tiling_on_sc=False)
_MESH = dict(
    mesh=plsc.VectorSubcoreMesh(core_axis_name="c", subcore_axis_name="s"),
    compiler_params=_SC_PARAMS,
)


def _deg_body(k_chunks, dstf, zeros1, ones, deg_out,
              idx_v, ones_v, stage_v, deg_sh, ssem):
    c = lax.axis_index("c")
    s = lax.axis_index("s")
    sl = pl.ds(s * RPT, RPT)
    pltpu.sync_copy(zeros1.at[sl], stage_v)
    pltpu.sync_copy(stage_v, deg_sh.at[sl])
    pltpu.sync_copy(ones, ones_v)
    plsc.subcore_barrier()
    w = c * NS + s
    base = w * k_chunks * CB

    def chunk(k, carry):
        pltpu.sync_copy(dstf.at[pl.ds(base + k * CB, CB)], idx_v)
        pltpu.async_copy(ones_v, deg_sh.at[idx_v], ssem, add=True)
        pltpu.make_async_copy(ones_v, deg_sh.at[idx_v], ssem).wait()
        return carry

    lax.fori_loop(0, k_chunks, chunk, 0)
    plsc.subcore_barrier()
    pltpu.sync_copy(deg_sh.at[sl], stage_v)
    pltpu.sync_copy(stage_v, deg_out.at[pl.ds(c * NPAD + s * RPT, RPT)])


def _agg_body(k_chunks, tab0, tab1, srcf, dstf, zeros1, out0, out1,
              sidx, didx, rows0, rows1, stage_v,
              tab0_sh, tab1_sh, acc0_sh, acc1_sh, gsem, ssem):
    c = lax.axis_index("c")
    s = lax.axis_index("s")
    sl = pl.ds(s * RPT, RPT)
    pltpu.sync_copy(tab0.at[sl], stage_v)
    pltpu.sync_copy(stage_v, tab0_sh.at[sl])
    pltpu.sync_copy(tab1.at[sl], stage_v)
    pltpu.sync_copy(stage_v, tab1_sh.at[sl])
    pltpu.sync_copy(zeros1.at[sl], stage_v)
    pltpu.sync_copy(stage_v, acc0_sh.at[sl])
    pltpu.sync_copy(stage_v, acc1_sh.at[sl])
    plsc.subcore_barrier()
    w = c * NS + s
    base = w * k_chunks * CB

    def chunk(k, carry):
        pltpu.sync_copy(srcf.at[pl.ds(base + k * CB, CB)], sidx)
        pltpu.sync_copy(dstf.at[pl.ds(base + k * CB, CB)], didx)
        pltpu.async_copy(tab0_sh.at[sidx], rows0, gsem)
        pltpu.async_copy(tab1_sh.at[sidx], rows1, gsem)
        pltpu.make_async_copy(tab0_sh.at[sidx], rows0, gsem).wait()
        pltpu.make_async_copy(tab1_sh.at[sidx], rows1, gsem).wait()
        pltpu.async_copy(rows0, acc0_sh.at[didx], ssem, add=True)
        pltpu.async_copy(rows1, acc1_sh.at[didx], ssem, add=True)
        pltpu.make_async_copy(rows0, acc0_sh.at[didx], ssem).wait()
        pltpu.make_async_copy(rows1, acc1_sh.at[didx], ssem).wait()
        return carry

    lax.fori_loop(0, k_chunks, chunk, 0)
    plsc.subcore_barrier()
    pltpu.sync_copy(acc0_sh.at[sl], stage_v)
    pltpu.sync_copy(stage_v, out0.at[pl.ds(c * NPAD + s * RPT, RPT)])
    pltpu.sync_copy(acc1_sh.at[sl], stage_v)
    pltpu.sync_copy(stage_v, out1.at[pl.ds(c * NPAD + s * RPT, RPT)])


def _sc_deg(dstf, zeros1, ones, k_chunks):
    return pl.kernel(
        functools.partial(_deg_body, k_chunks),
        out_type=jax.ShapeDtypeStruct((NC * NPAD,), jnp.float32),
        scratch_types=[
            pltpu.VMEM((CB,), jnp.int32),
            pltpu.VMEM((CB,), jnp.float32),
            pltpu.VMEM((RPT,), jnp.float32),
            pltpu.VMEM_SHARED((NPAD,), jnp.float32),
            pltpu.SemaphoreType.DMA,
        ],
        **_MESH)(dstf, zeros1, ones)


def _sc_agg(tab0, tab1, srcf, dstf, zeros1, k_chunks):
    return pl.kernel(
        functools.partial(_agg_body, k_chunks),
        out_type=[
            jax.ShapeDtypeStruct((NC * NPAD,), jnp.float32),
            jax.ShapeDtypeStruct((NC * NPAD,), jnp.float32),
        ],
        scratch_types=[
            pltpu.VMEM((CB,), jnp.int32),
            pltpu.VMEM((CB,), jnp.int32),
            pltpu.VMEM((CB,), jnp.float32),
            pltpu.VMEM((CB,), jnp.float32),
            pltpu.VMEM((RPT,), jnp.float32),
            pltpu.VMEM_SHARED((NPAD,), jnp.float32),
            pltpu.VMEM_SHARED((NPAD,), jnp.float32),
            pltpu.VMEM_SHARED((NPAD,), jnp.float32),
            pltpu.VMEM_SHARED((NPAD,), jnp.float32),
            pltpu.SemaphoreType.DMA,
            pltpu.SemaphoreType.DMA,
        ],
        **_MESH)(tab0, tab1, srcf, dstf, zeros1)


def _t1_body(degp, xt, dis, xs):
    deg = degp[0:1, :] + degp[1:2, :] + 1.0
    d = lax.rsqrt(deg)
    dis[...] = d
    xs[...] = xt[...] * d


def _t2_body(n, a0, a1, xs, dis, w1t, b1c, w2t, ys):
    i = pl.program_id(0)
    agg0 = a0[0:1, :] + a0[1:2, :] + xs[0:1, :]
    agg1 = a1[0:1, :] + a1[1:2, :] + xs[1:2, :]
    pre = jnp.concatenate([agg0, agg1], axis=0) * dis[...]
    h = jnp.dot(w1t[...], pre, preferred_element_type=jnp.float32) + b1c[...]
    h = jnp.maximum(h, 0.0)
    y = jnp.dot(w2t[...], h, preferred_element_type=jnp.float32)
    out = y * dis[...]
    col = i * TBL + lax.broadcasted_iota(jnp.int32, (2, TBL), 1)
    ys[...] = jnp.where(col < n, out, 0.0)


def _t3_body(a0, a1, ys, dis, b2c, out):
    v0 = a0[0:1, :] + a0[1:2, :] + ys[0:1, :]
    v1 = a1[0:1, :] + a1[1:2, :] + ys[1:2, :]
    v = jnp.concatenate([v0, v1], axis=0) * dis[...] + b2c[...]
    m = jnp.max(v, axis=0, keepdims=True)
    lse = m + jnp.log(jnp.sum(jnp.exp(v - m), axis=0, keepdims=True))
    out[...] = v - lse


def kernel(x, edge_index, W1, b1, W2, b2):
    n = x.shape[0]
    e = edge_index.shape[1]
    f_hid = W1.shape[1]
    assert n <= NPAD - 1 and NPAD % (NS * 8) == 0

    src = edge_index[0].astype(jnp.int32)
    dst = edge_index[1].astype(jnp.int32)
    step = NW * CB
    epad = ((e + step - 1) // step) * step
    if epad != e:
        fill = jnp.full((epad - e,), n, jnp.int32)
        src = jnp.concatenate([src, fill])
        dst = jnp.concatenate([dst, fill])
    k_chunks = epad // NW // CB

    xt = jnp.zeros((2, NPAD), jnp.float32).at[:, :n].set(x.T)
    zeros1 = jnp.zeros((NPAD,), jnp.float32)
    ones = jnp.ones((CB,), jnp.float32)

    degp = _sc_deg(dst, zeros1, ones, k_chunks)
    degp = degp.reshape(NC, NPAD)

    grid = NPAD // TBL
    dis, xs = pl.pallas_call(
        _t1_body,
        grid=(grid,),
        in_specs=[
            pl.BlockSpec((NC, TBL), lambda i: (0, i)),
            pl.BlockSpec((2, TBL), lambda i: (0, i)),
        ],
        out_specs=[
            pl.BlockSpec((1, TBL), lambda i: (0, i)),
            pl.BlockSpec((2, TBL), lambda i: (0, i)),
        ],
        out_shape=[
            jax.ShapeDtypeStruct((1, NPAD), jnp.float32),
            jax.ShapeDtypeStruct((2, NPAD), jnp.float32),
        ],
    )(degp, xt)

    a10, a11 = _sc_agg(xs[0], xs[1], src, dst, zeros1, k_chunks)

    ys = pl.pallas_call(
        functools.partial(_t2_body, n),
        grid=(grid,),
        in_specs=[
            pl.BlockSpec((NC, TBL), lambda i: (0, i)),
            pl.BlockSpec((NC, TBL), lambda i: (0, i)),
            pl.BlockSpec((2, TBL), lambda i: (0, i)),
            pl.BlockSpec((1, TBL), lambda i: (0, i)),
            pl.BlockSpec((f_hid, 2), lambda i: (0, 0)),
            pl.BlockSpec((f_hid, 1), lambda i: (0, 0)),
            pl.BlockSpec((2, f_hid), lambda i: (0, 0)),
        ],
        out_specs=pl.BlockSpec((2, TBL), lambda i: (0, i)),
        out_shape=jax.ShapeDtypeStruct((2, NPAD), jnp.float32),
    )(a10.reshape(NC, NPAD), a11.reshape(NC, NPAD), xs, dis,
      W1.T, b1.reshape(f_hid, 1), W2.T)

    a20, a21 = _sc_agg(ys[0], ys[1], src, dst, zeros1, k_chunks)

    out_t = pl.pallas_call(
        _t3_body,
        grid=(grid,),
        in_specs=[
            pl.BlockSpec((NC, TBL), lambda i: (0, i)),
            pl.BlockSpec((NC, TBL), lambda i: (0, i)),
            pl.BlockSpec((2, TBL), lambda i: (0, i)),
            pl.BlockSpec((1, TBL), lambda i: (0, i)),
            pl.BlockSpec((2, 1), lambda i: (0, 0)),
        ],
        out_specs=pl.BlockSpec((2, TBL), lambda i: (0, i)),
        out_shape=jax.ShapeDtypeStruct((2, NPAD), jnp.float32),
    )(a20.reshape(NC, NPAD), a21.reshape(NC, NPAD), ys, dis,
      b2.reshape(2, 1))

    return out_t[:, :n].T


# speedup vs baseline: 1.5450x; 1.5255x over previous
import functools

import jax
import jax.numpy as jnp
from jax import lax
from jax.experimental import pallas as pl
from jax.experimental.pallas import tpu as pltpu
from jax.experimental.pallas import tpu_sc as plsc

NC = 2
NS = 16
NW = NC * NS
SUB = 128
RC = 16
CHUNK = SUB * RC
NPAD = 102400
RPT = NPAD // NS
TBL = 2048

_SC_PARAMS = pltpu.CompilerParams(use_tc_tiling_on_sc=False)
_MESH = dict(
    mesh=plsc.VectorSubcoreMesh(core_axis_name="c", subcore_axis_name="s"),
    compiler_params=_SC_PARAMS,
)


def _deg_body(rows_w, k_chunks, dst_rows, zeros1, ones, deg_out,
              idx_v, ones_v, stage_v, deg_sh, ssem):
    c = lax.axis_index("c")
    s = lax.axis_index("s")
    sl = pl.ds(s * RPT, RPT)
    pltpu.sync_copy(zeros1.at[sl], stage_v)
    pltpu.sync_copy(stage_v, deg_sh.at[sl])
    pltpu.sync_copy(ones, ones_v)
    plsc.subcore_barrier()
    w = c * NS + s

    def chunk(k, carry):
        pltpu.sync_copy(dst_rows.at[pl.ds(w * rows_w + k * RC, RC)], idx_v)
        for r in range(RC):
            pltpu.async_copy(ones_v, deg_sh.at[idx_v.at[r]], ssem, add=True)
        for r in range(RC):
            pltpu.make_async_copy(ones_v, deg_sh.at[idx_v.at[r]], ssem).wait()
        return carry

    lax.fori_loop(0, k_chunks, chunk, 0)
    plsc.subcore_barrier()
    pltpu.sync_copy(deg_sh.at[sl], stage_v)
    pltpu.sync_copy(stage_v, deg_out.at[pl.ds(c * NPAD + s * RPT, RPT)])


def _agg_body(rows_w, k_chunks, tab0, tab1, src_rows, dst_rows, zeros1,
              out0, out1, sidx, didx, rows0, rows1, stage_v,
              tab0_sh, tab1_sh, acc0_sh, acc1_sh, gsem, ssem):
    c = lax.axis_index("c")
    s = lax.axis_index("s")
    sl = pl.ds(s * RPT, RPT)
    pltpu.sync_copy(tab0.at[sl], stage_v)
    pltpu.sync_copy(stage_v, tab0_sh.at[sl])
    pltpu.sync_copy(tab1.at[sl], stage_v)
    pltpu.sync_copy(stage_v, tab1_sh.at[sl])
    pltpu.sync_copy(zeros1.at[sl], stage_v)
    pltpu.sync_copy(stage_v, acc0_sh.at[sl])
    pltpu.sync_copy(stage_v, acc1_sh.at[sl])
    plsc.subcore_barrier()
    w = c * NS + s

    def chunk(k, carry):
        base = w * rows_w + k * RC
        pltpu.sync_copy(src_rows.at[pl.ds(base, RC)], sidx)
        pltpu.sync_copy(dst_rows.at[pl.ds(base, RC)], didx)
        for r in range(RC):
            pltpu.async_copy(tab0_sh.at[sidx.at[r]], rows0.at[r], gsem)
            pltpu.async_copy(tab1_sh.at[sidx.at[r]], rows1.at[r], gsem)
        for r in range(RC):
            pltpu.make_async_copy(tab0_sh.at[sidx.at[r]], rows0.at[r],
                                  gsem).wait()
            pltpu.make_async_copy(tab1_sh.at[sidx.at[r]], rows1.at[r],
                                  gsem).wait()
        for r in range(RC):
            pltpu.async_copy(rows0.at[r], acc0_sh.at[didx.at[r]], ssem,
                             add=True)
            pltpu.async_copy(rows1.at[r], acc1_sh.at[didx.at[r]], ssem,
                             add=True)
        for r in range(RC):
            pltpu.make_async_copy(rows0.at[r], acc0_sh.at[didx.at[r]],
                                  ssem).wait()
            pltpu.make_async_copy(rows1.at[r], acc1_sh.at[didx.at[r]],
                                  ssem).wait()
        return carry

    lax.fori_loop(0, k_chunks, chunk, 0)
    plsc.subcore_barrier()
    pltpu.sync_copy(acc0_sh.at[sl], stage_v)
    pltpu.sync_copy(stage_v, out0.at[pl.ds(c * NPAD + s * RPT, RPT)])
    pltpu.sync_copy(acc1_sh.at[sl], stage_v)
    pltpu.sync_copy(stage_v, out1.at[pl.ds(c * NPAD + s * RPT, RPT)])


def _sc_deg(dst_rows, zeros1, ones, rows_w, k_chunks):
    return pl.kernel(
        functools.partial(_deg_body, rows_w, k_chunks),
        out_type=jax.ShapeDtypeStruct((NC * NPAD,), jnp.float32),
        scratch_types=[
            pltpu.VMEM((RC, SUB), jnp.int32),
            pltpu.VMEM((SUB,), jnp.float32),
            pltpu.VMEM((RPT,), jnp.float32),
            pltpu.VMEM_SHARED((NPAD,), jnp.float32),
            pltpu.SemaphoreType.DMA,
        ],
        **_MESH)(dst_rows, zeros1, ones)


def _sc_agg(tab0, tab1, src_rows, dst_rows, zeros1, rows_w, k_chunks):
    return pl.kernel(
        functools.partial(_agg_body, rows_w, k_chunks),
        out_type=[
            jax.ShapeDtypeStruct((NC * NPAD,), jnp.float32),
            jax.ShapeDtypeStruct((NC * NPAD,), jnp.float32),
        ],
        scratch_types=[
            pltpu.VMEM((RC, SUB), jnp.int32),
            pltpu.VMEM((RC, SUB), jnp.int32),
            pltpu.VMEM((RC, SUB), jnp.float32),
            pltpu.VMEM((RC, SUB), jnp.float32),
            pltpu.VMEM((RPT,), jnp.float32),
            pltpu.VMEM_SHARED((NPAD,), jnp.float32),
            pltpu.VMEM_SHARED((NPAD,), jnp.float32),
            pltpu.VMEM_SHARED((NPAD,), jnp.float32),
            pltpu.VMEM_SHARED((NPAD,), jnp.float32),
            pltpu.SemaphoreType.DMA,
            pltpu.SemaphoreType.DMA,
        ],
        **_MESH)(tab0, tab1, src_rows, dst_rows, zeros1)


def _t1_body(degp, xt, dis, xs):
    deg = degp[0:1, :] + degp[1:2, :] + 1.0
    d = lax.rsqrt(deg)
    dis[...] = d
    xs[...] = xt[...] * d


def _t2_body(n, a0, a1, xs, dis, w1t, b1c, w2t, ys):
    i = pl.program_id(0)
    agg0 = a0[0:1, :] + a0[1:2, :] + xs[0:1, :]
    agg1 = a1[0:1, :] + a1[1:2, :] + xs[1:2, :]
    pre = jnp.concatenate([agg0, agg1], axis=0) * dis[...]
    h = jnp.dot(w1t[...], pre, preferred_element_type=jnp.float32) + b1c[...]
    h = jnp.maximum(h, 0.0)
    y = jnp.dot(w2t[...], h, preferred_element_type=jnp.float32)
    out = y * dis[...]
    col = i * TBL + lax.broadcasted_iota(jnp.int32, (2, TBL), 1)
    ys[...] = jnp.where(col < n, out, 0.0)


def _t3_body(a0, a1, ys, dis, b2c, out):
    v0 = a0[0:1, :] + a0[1:2, :] + ys[0:1, :]
    v1 = a1[0:1, :] + a1[1:2, :] + ys[1:2, :]
    v = jnp.concatenate([v0, v1], axis=0) * dis[...] + b2c[...]
    m = jnp.max(v, axis=0, keepdims=True)
    lse = m + jnp.log(jnp.sum(jnp.exp(v - m), axis=0, keepdims=True))
    out[...] = v - lse


def kernel(x, edge_index, W1, b1, W2, b2):
    n = x.shape[0]
    e = edge_index.shape[1]
    f_hid = W1.shape[1]
    assert n <= NPAD - 1 and NPAD % (NS * 8) == 0

    src = edge_index[0].astype(jnp.int32)
    dst = edge_index[1].astype(jnp.int32)
    step = NW * CHUNK
    epad = ((e + step - 1) // step) * step
    if epad != e:
        fill = jnp.full((epad - e,), n, jnp.int32)
        src = jnp.concatenate([src, fill])
        dst = jnp.concatenate([dst, fill])
    src_rows = src.reshape(epad // SUB, SUB)
    dst_rows = dst.reshape(epad // SUB, SUB)
    rows_w = epad // NW // SUB
    k_chunks = rows_w // RC

    xt = jnp.zeros((2, NPAD), jnp.float32).at[:, :n].set(x.T)
    zeros1 = jnp.zeros((NPAD,), jnp.float32)
    ones = jnp.ones((SUB,), jnp.float32)

    degp = _sc_deg(dst_rows, zeros1, ones, rows_w, k_chunks)
    degp = degp.reshape(NC, NPAD)

    grid = NPAD // TBL
    dis, xs = pl.pallas_call(
        _t1_body,
        grid=(grid,),
        in_specs=[
            pl.BlockSpec((NC, TBL), lambda i: (0, i)),
            pl.BlockSpec((2, TBL), lambda i: (0, i)),
        ],
        out_specs=[
            pl.BlockSpec((1, TBL), lambda i: (0, i)),
            pl.BlockSpec((2, TBL), lambda i: (0, i)),
        ],
        out_shape=[
            jax.ShapeDtypeStruct((1, NPAD), jnp.float32),
            jax.ShapeDtypeStruct((2, NPAD), jnp.float32),
        ],
    )(degp, xt)

    a10, a11 = _sc_agg(xs[0], xs[1], src_rows, dst_rows, zeros1,
                       rows_w, k_chunks)

    ys = pl.pallas_call(
        functools.partial(_t2_body, n),
        grid=(grid,),
        in_specs=[
            pl.BlockSpec((NC, TBL), lambda i: (0, i)),
            pl.BlockSpec((NC, TBL), lambda i: (0, i)),
            pl.BlockSpec((2, TBL), lambda i: (0, i)),
            pl.BlockSpec((1, TBL), lambda i: (0, i)),
            pl.BlockSpec((f_hid, 2), lambda i: (0, 0)),
            pl.BlockSpec((f_hid, 1), lambda i: (0, 0)),
            pl.BlockSpec((2, f_hid), lambda i: (0, 0)),
        ],
        out_specs=pl.BlockSpec((2, TBL), lambda i: (0, i)),
        out_shape=jax.ShapeDtypeStruct((2, NPAD), jnp.float32),
    )(a10.reshape(NC, NPAD), a11.reshape(NC, NPAD), xs, dis,
      W1.T, b1.reshape(f_hid, 1), W2.T)

    a20, a21 = _sc_agg(ys[0], ys[1], src_rows, dst_rows, zeros1,
                       rows_w, k_chunks)

    out_t = pl.pallas_call(
        _t3_body,
        grid=(grid,),
        in_specs=[
            pl.BlockSpec((NC, TBL), lambda i: (0, i)),
            pl.BlockSpec((NC, TBL), lambda i: (0, i)),
            pl.BlockSpec((2, TBL), lambda i: (0, i)),
            pl.BlockSpec((1, TBL), lambda i: (0, i)),
            pl.BlockSpec((2, 1), lambda i: (0, 0)),
        ],
        out_specs=pl.BlockSpec((2, TBL), lambda i: (0, i)),
        out_shape=jax.ShapeDtypeStruct((2, NPAD), jnp.float32),
    )(a20.reshape(NC, NPAD), a21.reshape(NC, NPAD), ys, dis,
      b2.reshape(2, 1))

    return out_t[:, :n].T


# speedup vs baseline: 1.7632x; 1.1412x over previous
import functools

import jax
import jax.numpy as jnp
from jax import lax
from jax.experimental import pallas as pl
from jax.experimental.pallas import tpu as pltpu
from jax.experimental.pallas import tpu_sc as plsc

NC = 2
NS = 16
NW = NC * NS
SUB = 128
RC = 32
CHUNK = SUB * RC
NPAD = 102400
RPT = NPAD // NS
TBL = 2048

_SC_PARAMS = pltpu.CompilerParams(use_tc_tiling_on_sc=False)
_MESH = dict(
    mesh=plsc.VectorSubcoreMesh(core_axis_name="c", subcore_axis_name="s"),
    compiler_params=_SC_PARAMS,
)


def _deg_body(rows_w, k_chunks, dst_rows, zeros1, ones, deg_out,
              idx_v, ones_v, stage_v, deg_sh, ssem):
    c = lax.axis_index("c")
    s = lax.axis_index("s")
    sl = pl.ds(s * RPT, RPT)
    pltpu.sync_copy(zeros1.at[sl], stage_v)
    pltpu.sync_copy(stage_v, deg_sh.at[sl])
    pltpu.sync_copy(ones, ones_v)
    plsc.subcore_barrier()
    w = c * NS + s

    def chunk(k, carry):
        pltpu.sync_copy(dst_rows.at[pl.ds(w * rows_w + k * RC, RC)], idx_v)
        for r in range(RC):
            pltpu.async_copy(ones_v, deg_sh.at[idx_v.at[r]], ssem, add=True)
        for r in range(RC):
            pltpu.make_async_copy(ones_v, deg_sh.at[idx_v.at[r]], ssem).wait()
        return carry

    lax.fori_loop(0, k_chunks, chunk, 0)
    plsc.subcore_barrier()
    pltpu.sync_copy(deg_sh.at[sl], stage_v)
    pltpu.sync_copy(stage_v, deg_out.at[pl.ds(c * NPAD + s * RPT, RPT)])


def _agg_body(rows_w, k_chunks, tab0, tab1, src_rows, dst_rows, zeros1,
              out0, out1, sidx, didx, rows0, rows1, stage_v,
              tab0_sh, tab1_sh, acc0_sh, acc1_sh, gsem, ssem):
    c = lax.axis_index("c")
    s = lax.axis_index("s")
    sl = pl.ds(s * RPT, RPT)
    pltpu.sync_copy(tab0.at[sl], stage_v)
    pltpu.sync_copy(stage_v, tab0_sh.at[sl])
    pltpu.sync_copy(tab1.at[sl], stage_v)
    pltpu.sync_copy(stage_v, tab1_sh.at[sl])
    pltpu.sync_copy(zeros1.at[sl], stage_v)
    pltpu.sync_copy(stage_v, acc0_sh.at[sl])
    pltpu.sync_copy(stage_v, acc1_sh.at[sl])
    plsc.subcore_barrier()
    w = c * NS + s

    def chunk(k, carry):
        base = w * rows_w + k * RC
        pltpu.sync_copy(src_rows.at[pl.ds(base, RC)], sidx)
        pltpu.sync_copy(dst_rows.at[pl.ds(base, RC)], didx)
        for r in range(RC):
            pltpu.async_copy(tab0_sh.at[sidx.at[r]], rows0.at[r], gsem)
            pltpu.async_copy(tab1_sh.at[sidx.at[r]], rows1.at[r], gsem)
        for r in range(RC):
            pltpu.make_async_copy(tab0_sh.at[sidx.at[r]], rows0.at[r],
                                  gsem).wait()
            pltpu.make_async_copy(tab1_sh.at[sidx.at[r]], rows1.at[r],
                                  gsem).wait()
        for r in range(RC):
            pltpu.async_copy(rows0.at[r], acc0_sh.at[didx.at[r]], ssem,
                             add=True)
            pltpu.async_copy(rows1.at[r], acc1_sh.at[didx.at[r]], ssem,
                             add=True)
        for r in range(RC):
            pltpu.make_async_copy(rows0.at[r], acc0_sh.at[didx.at[r]],
                                  ssem).wait()
            pltpu.make_async_copy(rows1.at[r], acc1_sh.at[didx.at[r]],
                                  ssem).wait()
        return carry

    lax.fori_loop(0, k_chunks, chunk, 0)
    plsc.subcore_barrier()
    pltpu.sync_copy(acc0_sh.at[sl], stage_v)
    pltpu.sync_copy(stage_v, out0.at[pl.ds(c * NPAD + s * RPT, RPT)])
    pltpu.sync_copy(acc1_sh.at[sl], stage_v)
    pltpu.sync_copy(stage_v, out1.at[pl.ds(c * NPAD + s * RPT, RPT)])


def _sc_deg(dst_rows, zeros1, ones, rows_w, k_chunks):
    return pl.kernel(
        functools.partial(_deg_body, rows_w, k_chunks),
        out_type=jax.ShapeDtypeStruct((NC * NPAD,), jnp.float32),
        scratch_types=[
            pltpu.VMEM((RC, SUB), jnp.int32),
            pltpu.VMEM((SUB,), jnp.float32),
            pltpu.VMEM((RPT,), jnp.float32),
            pltpu.VMEM_SHARED((NPAD,), jnp.float32),
            pltpu.SemaphoreType.DMA,
        ],
        **_MESH)(dst_rows, zeros1, ones)


def _sc_agg(tab0, tab1, src_rows, dst_rows, zeros1, rows_w, k_chunks):
    return pl.kernel(
        functools.partial(_agg_body, rows_w, k_chunks),
        out_type=[
            jax.ShapeDtypeStruct((NC * NPAD,), jnp.float32),
            jax.ShapeDtypeStruct((NC * NPAD,), jnp.float32),
        ],
        scratch_types=[
            pltpu.VMEM((RC, SUB), jnp.int32),
            pltpu.VMEM((RC, SUB), jnp.int32),
            pltpu.VMEM((RC, SUB), jnp.float32),
            pltpu.VMEM((RC, SUB), jnp.float32),
            pltpu.VMEM((RPT,), jnp.float32),
            pltpu.VMEM_SHARED((NPAD,), jnp.float32),
            pltpu.VMEM_SHARED((NPAD,), jnp.float32),
            pltpu.VMEM_SHARED((NPAD,), jnp.float32),
            pltpu.VMEM_SHARED((NPAD,), jnp.float32),
            pltpu.SemaphoreType.DMA,
            pltpu.SemaphoreType.DMA,
        ],
        **_MESH)(tab0, tab1, src_rows, dst_rows, zeros1)


def _t1_body(degp, xt, dis, xs):
    deg = degp[0:1, :] + degp[1:2, :] + 1.0
    d = lax.rsqrt(deg)
    dis[...] = d
    xs[...] = xt[...] * d


def _t2_body(n, a0, a1, xs, dis, w1t, b1c, w2t, ys):
    i = pl.program_id(0)
    agg0 = a0[0:1, :] + a0[1:2, :] + xs[0:1, :]
    agg1 = a1[0:1, :] + a1[1:2, :] + xs[1:2, :]
    pre = jnp.concatenate([agg0, agg1], axis=0) * dis[...]
    h = jnp.dot(w1t[...], pre, preferred_element_type=jnp.float32) + b1c[...]
    h = jnp.maximum(h, 0.0)
    y = jnp.dot(w2t[...], h, preferred_element_type=jnp.float32)
    out = y * dis[...]
    col = i * TBL + lax.broadcasted_iota(jnp.int32, (2, TBL), 1)
    ys[...] = jnp.where(col < n, out, 0.0)


def _t3_body(a0, a1, ys, dis, b2c, out):
    v0 = a0[0:1, :] + a0[1:2, :] + ys[0:1, :]
    v1 = a1[0:1, :] + a1[1:2, :] + ys[1:2, :]
    v = jnp.concatenate([v0, v1], axis=0) * dis[...] + b2c[...]
    m = jnp.max(v, axis=0, keepdims=True)
    lse = m + jnp.log(jnp.sum(jnp.exp(v - m), axis=0, keepdims=True))
    out[...] = v - lse


def kernel(x, edge_index, W1, b1, W2, b2):
    n = x.shape[0]
    e = edge_index.shape[1]
    f_hid = W1.shape[1]
    assert n <= NPAD - 1 and NPAD % (NS * 8) == 0

    src = edge_index[0].astype(jnp.int32)
    dst = edge_index[1].astype(jnp.int32)
    step = NW * CHUNK
    epad = ((e + step - 1) // step) * step
    if epad != e:
        fill = jnp.full((epad - e,), n, jnp.int32)
        src = jnp.concatenate([src, fill])
        dst = jnp.concatenate([dst, fill])
    src_rows = src.reshape(epad // SUB, SUB)
    dst_rows = dst.reshape(epad // SUB, SUB)
    rows_w = epad // NW // SUB
    k_chunks = rows_w // RC

    xt = jnp.zeros((2, NPAD), jnp.float32).at[:, :n].set(x.T)
    zeros1 = jnp.zeros((NPAD,), jnp.float32)
    ones = jnp.ones((SUB,), jnp.float32)

    degp = _sc_deg(dst_rows, zeros1, ones, rows_w, k_chunks)
    degp = degp.reshape(NC, NPAD)

    grid = NPAD // TBL
    dis, xs = pl.pallas_call(
        _t1_body,
        grid=(grid,),
        in_specs=[
            pl.BlockSpec((NC, TBL), lambda i: (0, i)),
            pl.BlockSpec((2, TBL), lambda i: (0, i)),
        ],
        out_specs=[
            pl.BlockSpec((1, TBL), lambda i: (0, i)),
            pl.BlockSpec((2, TBL), lambda i: (0, i)),
        ],
        out_shape=[
            jax.ShapeDtypeStruct((1, NPAD), jnp.float32),
            jax.ShapeDtypeStruct((2, NPAD), jnp.float32),
        ],
    )(degp, xt)

    a10, a11 = _sc_agg(xs[0], xs[1], src_rows, dst_rows, zeros1,
                       rows_w, k_chunks)

    ys = pl.pallas_call(
        functools.partial(_t2_body, n),
        grid=(grid,),
        in_specs=[
            pl.BlockSpec((NC, TBL), lambda i: (0, i)),
            pl.BlockSpec((NC, TBL), lambda i: (0, i)),
            pl.BlockSpec((2, TBL), lambda i: (0, i)),
            pl.BlockSpec((1, TBL), lambda i: (0, i)),
            pl.BlockSpec((f_hid, 2), lambda i: (0, 0)),
            pl.BlockSpec((f_hid, 1), lambda i: (0, 0)),
            pl.BlockSpec((2, f_hid), lambda i: (0, 0)),
        ],
        out_specs=pl.BlockSpec((2, TBL), lambda i: (0, i)),
        out_shape=jax.ShapeDtypeStruct((2, NPAD), jnp.float32),
    )(a10.reshape(NC, NPAD), a11.reshape(NC, NPAD), xs, dis,
      W1.T, b1.reshape(f_hid, 1), W2.T)

    a20, a21 = _sc_agg(ys[0], ys[1], src_rows, dst_rows, zeros1,
                       rows_w, k_chunks)

    out_t = pl.pallas_call(
        _t3_body,
        grid=(grid,),
        in_specs=[
            pl.BlockSpec((NC, TBL), lambda i: (0, i)),
            pl.BlockSpec((NC, TBL), lambda i: (0, i)),
            pl.BlockSpec((2, TBL), lambda i: (0, i)),
            pl.BlockSpec((1, TBL), lambda i: (0, i)),
            pl.BlockSpec((2, 1), lambda i: (0, 0)),
        ],
        out_specs=pl.BlockSpec((2, TBL), lambda i: (0, i)),
        out_shape=jax.ShapeDtypeStruct((2, NPAD), jnp.float32),
    )(a20.reshape(NC, NPAD), a21.reshape(NC, NPAD), ys, dis,
      b2.reshape(2, 1))

    return out_t[:, :n].T
